# Initial kernel scaffold; baseline (speedup 1.0000x reference)
import jax
import jax.numpy as jnp
from jax.experimental import pallas as pl


def kernel(t, h, x, edge_index, params):
    raise NotImplementedError("write your pallas kernel here")


# speedup vs baseline: 8.6320x; 8.6320x over previous
import functools

import numpy as np
import jax
import jax.numpy as jnp
from jax import lax
from jax.experimental import pallas as pl
from jax.experimental.pallas import tpu as pltpu

CR = 10.0


def _swish(x, beta):
    return x * jax.nn.sigmoid(beta * x)


def _gvp_flat(s, vflat, nv, p):
    Vh = [vflat[:, d * nv:(d + 1) * nv] @ p['Wh'] for d in range(3)]
    vn = jnp.sqrt(Vh[0] ** 2 + Vh[1] ** 2 + Vh[2] ** 2 + 1e-8)
    s_lin = jnp.concatenate([s, vn], axis=-1) @ p['Ws'] + p['bs']
    s_out = _swish(s_lin, p['beta'])
    gate = jax.nn.sigmoid(s_out @ p['Wg'] + p['bg'])
    outv = jnp.concatenate([(Vh[d] @ p['Wu']) * gate for d in range(3)], axis=-1)
    return s_out, outv


def _edge_fn(eb, xdb, p):
    hs = eb[:, 0:64]
    xs = eb[:, 112:115]
    xd = xdb[:, 0:3]
    dx = xd - xs
    dist = jnp.sqrt(jnp.sum(dx * dx, axis=-1, keepdims=True) + 1e-8)
    dirv = dx / dist
    vin = jnp.concatenate(
        [jnp.concatenate([eb[:, 64 + 16 * d:64 + 16 * (d + 1)], dirv[:, d:d + 1]],
                         axis=-1) for d in range(3)], axis=-1)
    s_in = jnp.concatenate([hs, dist], axis=-1)
    m_s, outv = _gvp_flat(s_in, vin, 17, p)
    m_v = jnp.concatenate([outv[:, d * 17:d * 17 + 16] for d in range(3)], axis=-1)
    m_x = jnp.stack([outv[:, d * 17 + 16] for d in range(3)], axis=-1)
    return m_s, m_v, m_x


def _edge_fwd_impl(tab, xtab, src, dst, p):
    n = tab.shape[0]
    eb = tab[src]
    xdb = xtab[dst]
    m_s, m_v, m_x = _edge_fn(eb, xdb, p)
    m = jnp.concatenate([m_s, m_v, m_x], axis=-1)
    sums = jax.ops.segment_sum(m, dst, num_segments=n)
    return sums[:, 0:64], sums[:, 64:112], sums[:, 112:115], (eb, xdb)


def _edge_bwd_impl(res, src, dst, p, gs, gv, gx):
    eb, xdb = res
    n = res[0].shape[0] if False else None
    nnodes = gs.shape[0]
    gme = jnp.concatenate([gs, gv, gx], axis=-1)[dst]
    _, vjp_fn = jax.vjp(lambda e_, xd_: _edge_fn(e_, xd_, p), eb, xdb)
    eb_bar, xd_bar = vjp_fn((gme[:, 0:64], gme[:, 64:112], gme[:, 112:115]))
    by_src = jax.ops.segment_sum(eb_bar[:, 0:115], src, num_segments=nnodes)
    by_dst = jax.ops.segment_sum(xd_bar[:, 0:3], dst, num_segments=nnodes)
    h_bar = by_src[:, 0:64]
    v_bar = by_src[:, 64:112]
    x_bar = by_src[:, 112:115] + by_dst
    return h_bar, x_bar, v_bar


def _make_edge_aggregate():
    @jax.custom_vjp
    def agg(h, x, v2, src, dst, p):
        tab = jnp.concatenate(
            [h, v2, x, jnp.zeros((h.shape[0], 13), h.dtype)], axis=-1)
        xtab = jnp.concatenate([x, jnp.zeros((x.shape[0], 13), x.dtype)], axis=-1)
        ss, sv, sx, _ = _edge_fwd_impl(tab, xtab, src, dst, p)
        return ss, sv, sx

    def agg_fwd(h, x, v2, src, dst, p):
        tab = jnp.concatenate(
            [h, v2, x, jnp.zeros((h.shape[0], 13), h.dtype)], axis=-1)
        xtab = jnp.concatenate([x, jnp.zeros((x.shape[0], 13), x.dtype)], axis=-1)
        ss, sv, sx, res = _edge_fwd_impl(tab, xtab, src, dst, p)
        return (ss, sv, sx), (res, src, dst, p)

    def agg_bwd(saved, g):
        res, src, dst, p = saved
        gs, gv, gx = g
        h_bar, x_bar, v_bar = _edge_bwd_impl(res, src, dst, p, gs, gv, gx)
        f0 = np.zeros(src.shape, jax.dtypes.float0)
        p_zeros = jax.tree.map(jnp.zeros_like, p)
        return (h_bar, x_bar, v_bar, f0, f0, p_zeros)

    agg.defvjp(agg_fwd, agg_bwd)
    return agg


_edge_aggregate = _make_edge_aggregate()


def _forward(t, h0, x0, src, dst, inv_cnt, params, nrep, ngraph):
    n = h0.shape[0]
    ts = jnp.repeat(t, nrep).reshape(-1, 1)
    z = jnp.concatenate([h0, ts], axis=-1)
    h = jax.nn.silu(z @ params['emb_w'] + params['emb_b'])
    v2 = jnp.zeros((n, 48), h.dtype)
    x = x0
    for lp in params['layers']:
        ss, sv, sx, = _edge_aggregate(h, x, v2, src, dst, lp['msg'])
        agg_s = ss * inv_cnt
        agg_v = sv * inv_cnt
        agg_x = sx * inv_cnt
        x_out = x + CR * jnp.tanh(agg_x)
        s_in = jnp.concatenate([h, agg_s], axis=-1)
        v_in = jnp.concatenate(
            [jnp.concatenate([v2[:, 16 * d:16 * (d + 1)],
                              agg_v[:, 16 * d:16 * (d + 1)]], axis=-1)
             for d in range(3)], axis=-1)
        u_s, u_v = _gvp_flat(s_in, v_in, 32, lp['upd'])
        h = h + u_s
        v2 = v2 + u_v
        x = x_out
    pooled = h.reshape(ngraph, nrep, h.shape[-1]).mean(axis=1)
    e = jax.nn.silu(pooled @ params['out_w1'] + params['out_b1']) @ params['out_w2'] + params['out_b2']
    return e


def kernel(t, h, x, edge_index, params):
    n = h.shape[0]
    ngraph = t.shape[0]
    nrep = n // ngraph
    src = edge_index[0]
    dst = edge_index[1]
    cnt = jax.ops.segment_sum(jnp.ones((src.shape[0],), jnp.float32), dst,
                              num_segments=n)
    inv_cnt = (1.0 / jnp.maximum(cnt, 1.0))[:, None]

    def f(xx):
        e = _forward(t, h, xx, src, dst, inv_cnt, params, nrep, ngraph)
        return e.sum(), e

    position_grad, energy = jax.grad(f, has_aux=True)(x)
    return (position_grad, energy)


# speedup vs baseline: 14.3432x; 1.6616x over previous
import functools

import numpy as np
import jax
import jax.numpy as jnp
from jax import lax
from jax.experimental import pallas as pl
from jax.experimental.pallas import tpu as pltpu
from jax.experimental.pallas import tpu_sc as plsc

CR = 10.0
NW = 32
BE = 2048
SBLK = 2048
GBLK = 128


def _swish(x, beta):
    return x * jax.nn.sigmoid(beta * x)


def _gvp_flat(s, vflat, nv, p):
    Vh = [vflat[:, d * nv:(d + 1) * nv] @ p['Wh'] for d in range(3)]
    vn = jnp.sqrt(Vh[0] ** 2 + Vh[1] ** 2 + Vh[2] ** 2 + 1e-8)
    s_lin = jnp.concatenate([s, vn], axis=-1) @ p['Ws'] + p['bs']
    s_out = _swish(s_lin, p['beta'])
    gate = jax.nn.sigmoid(s_out @ p['Wg'] + p['bg'])
    outv = jnp.concatenate([(Vh[d] @ p['Wu']) * gate for d in range(3)], axis=-1)
    return s_out, outv


def _edge_fn(eb, xdb, p):
    hs = eb[:, 0:64]
    xs = eb[:, 112:115]
    xd = xdb[:, 112:115]
    dx = xd - xs
    dist = jnp.sqrt(jnp.sum(dx * dx, axis=-1, keepdims=True) + 1e-8)
    dirv = dx / dist
    vin = jnp.concatenate(
        [jnp.concatenate([eb[:, 64 + 16 * d:64 + 16 * (d + 1)], dirv[:, d:d + 1]],
                         axis=-1) for d in range(3)], axis=-1)
    s_in = jnp.concatenate([hs, dist], axis=-1)
    m_s, outv = _gvp_flat(s_in, vin, 17, p)
    m_v = jnp.concatenate([outv[:, d * 17:d * 17 + 16] for d in range(3)], axis=-1)
    m_x = jnp.stack([outv[:, d * 17 + 16] for d in range(3)], axis=-1)
    return jnp.concatenate([m_s, m_v, m_x], axis=-1)


def _wlist(p):
    return [p['Wh'], p['Ws'], p['bs'].reshape(1, -1), p['Wu'], p['Wg'],
            p['bg'].reshape(1, -1), p['beta'].reshape(1, 1)]


def _wdict(refs):
    Wh, Ws, bs, Wu, Wg, bg, beta = [r[...] for r in refs]
    return {'Wh': Wh, 'Ws': Ws, 'bs': bs[0], 'Wu': Wu, 'Wg': Wg, 'bg': bg[0],
            'beta': beta[0, 0]}


def _tc_fwd_body(eb_ref, xdb_ref, *rest):
    out_ref = rest[-1]
    p = _wdict(rest[:-1])
    m = _edge_fn(eb_ref[...], xdb_ref[...], p)
    b = m.shape[0]
    full = jnp.concatenate(
        [m, jnp.ones((b, 1), m.dtype), jnp.zeros((b, 12), m.dtype)], axis=-1)
    out_ref[...] = full.T


def _tc_bwd_body(eb_ref, xdb_ref, g_ref, *rest):
    out_ref = rest[-1]
    p = _wdict(rest[:-1])
    eb = eb_ref[...]
    xdb = xdb_ref[...]
    g = g_ref[...]
    _, vjp_fn = jax.vjp(lambda e_, xd_: _edge_fn(e_, xd_, p), eb, xdb)
    eb_bar, xd_bar = vjp_fn(g[:, 0:115])
    b = eb.shape[0]
    full = jnp.concatenate(
        [eb_bar[:, 0:115], jnp.zeros((b, 1), eb.dtype), xd_bar[:, 112:115],
         jnp.zeros((b, 9), eb.dtype)], axis=-1)
    out_ref[...] = full.T


def _tc_edge(body, eb, xdb, gme, p):
    ep = eb.shape[0]
    grid = (ep // BE,)
    wl = _wlist(p)
    ins = [eb, xdb] + ([gme] if gme is not None else []) + wl
    specs = [pl.BlockSpec((BE, 128), lambda i: (i, 0)),
             pl.BlockSpec((BE, 128), lambda i: (i, 0))]
    if gme is not None:
        specs.append(pl.BlockSpec((BE, 128), lambda i: (i, 0)))
    for w in wl:
        specs.append(pl.BlockSpec(w.shape, lambda i, nd=w.ndim: (0,) * nd))
    return pl.pallas_call(
        body,
        grid=grid,
        in_specs=specs,
        out_specs=pl.BlockSpec((128, BE), lambda i: (0, i)),
        out_shape=jax.ShapeDtypeStruct((128, ep), jnp.float32),
    )(*ins)


def _tc_edge_fwd(eb, xdb, p):
    return _tc_edge(_tc_fwd_body, eb, xdb, None, p)


def _tc_edge_bwd(eb, xdb, gme, p):
    return _tc_edge(_tc_bwd_body, eb, xdb, gme, p)


@functools.partial(jax.jit, static_argnames=('d',))
def _sc_gather(tab, idx, d):
    ep = idx.shape[0]
    per_w = ep // NW
    steps = per_w // GBLK
    mesh = plsc.VectorSubcoreMesh(core_axis_name="c", subcore_axis_name="s")

    @functools.partial(
        pl.kernel, mesh=mesh,
        out_type=jax.ShapeDtypeStruct((ep, d), jnp.float32),
        scratch_types=[
            pltpu.VMEM((per_w,), jnp.int32),
            pltpu.VMEM((GBLK, d), jnp.float32),
            pltpu.VMEM((GBLK, d), jnp.float32),
            pltpu.SemaphoreType.DMA,
            pltpu.SemaphoreType.DMA,
            pltpu.SemaphoreType.DMA,
        ],
    )
    def k(tab_hbm, idx_hbm, out_hbm, idxv, rows0, rows1, gsem, wsem0, wsem1):
        wid = lax.axis_index("s") * 2 + lax.axis_index("c")
        base = wid * per_w
        pltpu.sync_copy(idx_hbm.at[pl.ds(base, per_w)], idxv)

        def gather_into(s, rbuf):
            pltpu.async_copy(tab_hbm.at[idxv.at[pl.ds(s * GBLK, GBLK)]], rbuf,
                             gsem).wait()

        def put(s, rbuf, wsem):
            pltpu.async_copy(rbuf, out_hbm.at[pl.ds(base + s * GBLK, GBLK)],
                             wsem)

        def wait_put(rbuf, wsem):
            pltpu.make_async_copy(rbuf, out_hbm.at[pl.ds(0, GBLK)], wsem).wait()

        def body(s, _):
            def do(rbuf, wsem):
                @pl.when(s >= 2)
                def _():
                    wait_put(rbuf, wsem)
                gather_into(s, rbuf)
                put(s, rbuf, wsem)

            @pl.when(lax.rem(s, 2) == 0)
            def _():
                do(rows0, wsem0)

            @pl.when(lax.rem(s, 2) == 1)
            def _():
                do(rows1, wsem1)
            return _

        lax.fori_loop(0, steps, body, None)
        wait_put(rows0, wsem0)
        if steps >= 2:
            wait_put(rows1, wsem1)

    return k(tab, idx)


@functools.partial(jax.jit, static_argnames=('nnodes', 'nrows_out', 'nrows_a'))
def _sc_scatter(vals, idx2, nnodes, nrows_out, nrows_a):
    nrows, ep = vals.shape
    nnp = ((nnodes + 127) // 128) * 128
    nchunk = nrows_out // 2
    rounds = (nchunk + NW - 1) // NW
    nblk = ep // SBLK
    mesh = plsc.VectorSubcoreMesh(core_axis_name="c", subcore_axis_name="s")

    @functools.partial(
        pl.kernel, mesh=mesh,
        compiler_params=pltpu.CompilerParams(needs_layout_passes=False),
        out_type=jax.ShapeDtypeStruct((nrows_out, nnp), jnp.float32),
        scratch_types=[
            pltpu.VMEM((2 * nnp,), jnp.float32),
            pltpu.VMEM((2, SBLK), jnp.int32),
            pltpu.VMEM((2, 2, SBLK), jnp.float32),
            pltpu.SemaphoreType.DMA,
            pltpu.SemaphoreType.DMA,
            pltpu.SemaphoreType.DMA,
        ],
    )
    def k(vals_hbm, idx2_hbm, out_hbm, acc, idxb, valb, sem0, sem1, osem):
        wid = lax.axis_index("s") * 2 + lax.axis_index("c")

        def start_blk(b, j, rowsel, r0, sem):
            pltpu.async_copy(idx2_hbm.at[rowsel, pl.ds(b * SBLK, SBLK)],
                             idxb.at[j], sem)
            pltpu.async_copy(vals_hbm.at[r0, pl.ds(b * SBLK, SBLK)],
                             valb.at[j, 0], sem)
            pltpu.async_copy(vals_hbm.at[r0 + 1, pl.ds(b * SBLK, SBLK)],
                             valb.at[j, 1], sem)

        def wait_blk(j, sem):
            pltpu.make_async_copy(idx2_hbm.at[0, pl.ds(0, SBLK)], idxb.at[j],
                                  sem).wait()
            pltpu.make_async_copy(vals_hbm.at[0, pl.ds(0, SBLK)], valb.at[j, 0],
                                  sem).wait()
            pltpu.make_async_copy(vals_hbm.at[0, pl.ds(0, SBLK)], valb.at[j, 1],
                                  sem).wait()

        for r in range(rounds):
            chunk = wid + r * NW

            @pl.when(chunk < nchunk)
            def _():
                rowsel = (chunk >= (nrows_a // 2)).astype(jnp.int32)
                r0 = 2 * chunk

                def zbody(z, _):
                    acc[pl.ds(z * 16, 16)] = jnp.zeros((16,), jnp.float32)
                    return _
                lax.fori_loop(0, (2 * nnp) // 16, zbody, None)

                start_blk(0, 0, rowsel, r0, sem0)

                def blk_body(b, _):
                    j = lax.rem(b, 2)

                    def do(j_static, sem, osem_):
                        wait_blk(j_static, sem)

                        @pl.when(b + 1 < nblk)
                        def _():
                            start_blk(b + 1, 1 - j_static, rowsel, r0, osem_)

                        def inner(t, _):
                            off = t * 16
                            ii = idxb[j_static, pl.ds(off, 16)]
                            msk = ii < nnodes
                            v0 = valb[j_static, 0, pl.ds(off, 16)]
                            v1 = valb[j_static, 1, pl.ds(off, 16)]
                            plsc.addupdate_scatter(acc, [ii], v0, mask=msk)
                            plsc.addupdate_scatter(acc, [ii + nnp], v1,
                                                   mask=msk)
                            return _
                        lax.fori_loop(0, SBLK // 16, inner, None)

                    @pl.when(j == 0)
                    def _():
                        do(0, sem0, sem1)

                    @pl.when(j == 1)
                    def _():
                        do(1, sem1, sem0)
                    return _

                lax.fori_loop(0, nblk, blk_body, None)
                pltpu.async_copy(acc.at[pl.ds(0, nnp)], out_hbm.at[r0], osem)
                pltpu.async_copy(acc.at[pl.ds(nnp, nnp)],
                                 out_hbm.at[r0 + 1], osem)
                pltpu.make_async_copy(acc.at[pl.ds(0, nnp)], out_hbm.at[r0],
                                      osem).wait()
                pltpu.make_async_copy(acc.at[pl.ds(0, nnp)], out_hbm.at[r0],
                                      osem).wait()

    return k(vals, idx2)[:, :nnodes]


def _pack_tab(h, v2, x):
    n = h.shape[0]
    return jnp.concatenate([h, v2, x, jnp.zeros((n, 13), h.dtype)], axis=-1)


def _edge_stage_fwd_impl(h, x, v2, src_c, dst_c, dst2_s, p):
    n = h.shape[0]
    tab = _pack_tab(h, v2, x)
    eb = _sc_gather(tab, src_c, 128)
    xdb = _sc_gather(tab, dst_c, 128)
    mT = _tc_edge_fwd(eb, xdb, p)
    sums = _sc_scatter(mT, dst2_s, n, 116, 116)
    return sums, (eb, xdb)


@jax.custom_vjp
def _edge_stage(h, x, v2, src_c, dst_c, dst2_s, sd2_s, p):
    sums, _ = _edge_stage_fwd_impl(h, x, v2, src_c, dst_c, dst2_s, p)
    return sums


def _edge_stage_fwd(h, x, v2, src_c, dst_c, dst2_s, sd2_s, p):
    sums, res = _edge_stage_fwd_impl(h, x, v2, src_c, dst_c, dst2_s, p)
    return sums, (res, dst_c, sd2_s, p, h.shape[0])


def _edge_stage_bwd(saved, g):
    (eb, xdb), dst_c, sd2_s, p, n = saved
    gtab = jnp.concatenate(
        [g[0:115], jnp.zeros((13, n), g.dtype)], axis=0).T
    gme = _sc_gather(gtab, dst_c, 128)
    gradT = _tc_edge_bwd(eb, xdb, gme, p)
    by = _sc_scatter(gradT, sd2_s, n, 120, 116)
    h_bar = by[0:64].T
    v_bar = by[64:112].T
    x_bar = by[112:115].T + by[116:119].T
    f0 = np.zeros((sd2_s.shape[-1],), jax.dtypes.float0)
    f02 = np.zeros(sd2_s.shape, jax.dtypes.float0)
    p_zeros = jax.tree.map(jnp.zeros_like, p)
    return (h_bar, x_bar, v_bar, f0, f0, f02, f02, p_zeros)


_edge_stage.defvjp(_edge_stage_fwd, _edge_stage_bwd)


def _forward(t, h0, x0, src_c, dst_c, dst2_s, sd2_s, params, nrep, ngraph):
    n = h0.shape[0]
    ts = jnp.repeat(t, nrep).reshape(-1, 1)
    z = jnp.concatenate([h0, ts], axis=-1)
    h = jax.nn.silu(z @ params['emb_w'] + params['emb_b'])
    v2 = jnp.zeros((n, 48), h.dtype)
    x = x0
    inv_cnt = None
    for lp in params['layers']:
        sums = _edge_stage(h, x, v2, src_c, dst_c, dst2_s, sd2_s, lp['msg'])
        if inv_cnt is None:
            cnt = lax.stop_gradient(sums[115])
            inv_cnt = (1.0 / jnp.maximum(cnt, 1.0))[:, None]
        agg_s = sums[0:64].T * inv_cnt
        agg_v = sums[64:112].T * inv_cnt
        agg_x = sums[112:115].T * inv_cnt
        x_out = x + CR * jnp.tanh(agg_x)
        s_in = jnp.concatenate([h, agg_s], axis=-1)
        v_in = jnp.concatenate(
            [jnp.concatenate([v2[:, 16 * d:16 * (d + 1)],
                              agg_v[:, 16 * d:16 * (d + 1)]], axis=-1)
             for d in range(3)], axis=-1)
        u_s, u_v = _gvp_flat(s_in, v_in, 32, lp['upd'])
        h = h + u_s
        v2 = v2 + u_v
        x = x_out
    pooled = h.reshape(ngraph, nrep, h.shape[-1]).mean(axis=1)
    e = jax.nn.silu(pooled @ params['out_w1'] + params['out_b1']) \
        @ params['out_w2'] + params['out_b2']
    return e


def kernel(t, h, x, edge_index, params):
    n = h.shape[0]
    ngraph = t.shape[0]
    nrep = n // ngraph
    e = edge_index.shape[1]
    ep = ((e + 4095) // 4096) * 4096
    pad = ep - e
    src = edge_index[0]
    dst = edge_index[1]
    zpad = jnp.zeros((pad,), jnp.int32)
    npad = jnp.full((pad,), n, jnp.int32)
    src_c = jnp.concatenate([src, zpad])
    dst_c = jnp.concatenate([dst, zpad])
    src_s = jnp.concatenate([src, npad])
    dst_s = jnp.concatenate([dst, npad])
    dst2_s = jnp.stack([dst_s, dst_s])
    sd2_s = jnp.stack([src_s, dst_s])

    def f(xx):
        e_ = _forward(t, h, xx, src_c, dst_c, dst2_s, sd2_s, params, nrep,
                      ngraph)
        return e_.sum(), e_

    position_grad, energy = jax.grad(f, has_aux=True)(x)
    return (position_grad, energy)
